# baseline (speedup 1.0000x reference)
import functools

import jax
import jax.numpy as jnp
from jax import lax
from jax.experimental import pallas as pl
from jax.experimental.pallas import tpu as pltpu
from jax.experimental.pallas import tpu_sc as plsc

D_MODEL = 2048
D_HIDDEN = 4096
N_EXPERTS = 8
CAP = 640
B_TOK = 2048
N_SLOTS = N_EXPERTS * CAP

LOG_PAD = 128
BM_ENC = 256
CHUNK_I = 512
BH = 1024

NC, NS = 2, 16
NW = NC * NS
SC_CHUNK = 16


def _enc_body(x_ref, we_ref, be_ref, wg_ref, bg_ref, enc_ref, log_ref):
    enc = lax.dot_general(
        x_ref[...], we_ref[...], (((1,), (1,)), ((), ())),
        preferred_element_type=jnp.float32,
    ) + be_ref[...]
    enc_ref[...] = enc
    log_ref[...] = lax.dot_general(
        enc, wg_ref[...], (((1,), (1,)), ((), ())),
        preferred_element_type=jnp.float32,
    ) + bg_ref[...]


def _route_body(log_ref, pidx_ref, g1_ref, g2_ref, w1_ref, w2_ref):
    logits = log_ref[...]
    col = lax.broadcasted_iota(jnp.int32, (B_TOK, LOG_PAD), 1)
    v1 = jnp.max(logits, axis=1, keepdims=True)
    e1 = jnp.min(jnp.where(logits == v1, col, LOG_PAD), axis=1, keepdims=True)
    l2 = jnp.where(col == e1, -jnp.inf, logits)
    v2 = jnp.max(l2, axis=1, keepdims=True)
    e2 = jnp.min(jnp.where(l2 == v2, col, LOG_PAD), axis=1, keepdims=True)
    t = jnp.exp(v2 - v1)
    denom = (1.0 + t) + 1e-12
    w1 = 1.0 / denom
    w2 = t / denom

    eids = lax.broadcasted_iota(jnp.int32, (B_TOK, N_EXPERTS), 1)
    w_t = jnp.where(eids == e1, w1, 0.0) + jnp.where(eids == e2, w2, 0.0)
    w_all = jnp.transpose(w_t)
    j_row = lax.broadcasted_iota(jnp.int32, (1, B_TOK), 1)
    s_row = lax.broadcasted_iota(jnp.int32, (1, CAP), 1)

    slot1 = jnp.zeros((B_TOK, 1), jnp.int32)
    slot2 = jnp.zeros((B_TOK, 1), jnp.int32)
    k1 = jnp.zeros((B_TOK, 1), jnp.bool_)
    k2 = jnp.zeros((B_TOK, 1), jnp.bool_)

    for e in range(N_EXPERTS):
        w_row = w_all[e:e + 1, :]
        posr = w_row > 0.0
        cnt_chunks = []
        for c in range(B_TOK // CHUNK_I):
            i0 = c * CHUNK_I
            w_col = w_t[i0:i0 + CHUNK_I, e:e + 1]
            i_col = lax.broadcasted_iota(jnp.int32, (CHUNK_I, 1), 0) + i0
            beats = (w_row > w_col) | ((w_row == w_col) & (j_row < i_col))
            cnt_chunks.append(jnp.sum(
                jnp.where(posr & beats, 1, 0), axis=1, keepdims=True))
        cnt_e = jnp.concatenate(cnt_chunks, axis=0)
        kept_e = (w_t[:, e:e + 1] > 0.0) & (cnt_e < CAP)
        is1 = e1 == e
        is2 = e2 == e
        slot1 = jnp.where(is1, cnt_e, slot1)
        slot2 = jnp.where(is2, cnt_e, slot2)
        k1 = k1 | (is1 & kept_e)
        k2 = k2 | (is2 & kept_e)
        acc = jnp.zeros((1, CAP), jnp.int32)
        for c in range(B_TOK // CHUNK_I):
            i0 = c * CHUNK_I
            hit = kept_e[i0:i0 + CHUNK_I, :] & (cnt_e[i0:i0 + CHUNK_I, :] == s_row)
            i_col = lax.broadcasted_iota(jnp.int32, (CHUNK_I, 1), 0) + i0
            acc = acc + jnp.sum(jnp.where(hit, i_col, 0), axis=0, keepdims=True)
        pidx_ref[e:e + 1, :] = acc

    g1_ref[...] = jnp.where(k1, e1 * CAP + slot1, 0)
    g2_ref[...] = jnp.where(k2, e2 * CAP + slot2, 0)
    w1_ref[...] = jnp.where(k1, w1, 0.0)
    w2_ref[...] = jnp.where(k2, w2, 0.0)


def _ffn_body(xg_ref, w1_ref, w2_ref, out_ref):
    hb = pl.program_id(1)
    xb = xg_ref[0].astype(jnp.bfloat16)
    h = lax.dot_general(xb, w1_ref[0], (((1,), (1,)), ((), ())),
                        preferred_element_type=jnp.float32)
    h = jnp.maximum(h, 0.0).astype(jnp.bfloat16)
    part = lax.dot_general(h, w2_ref[0], (((1,), (1,)), ((), ())),
                           preferred_element_type=jnp.float32)

    @pl.when(hb == 0)
    def _():
        out_ref[0] = part

    @pl.when(hb != 0)
    def _():
        out_ref[0] += part


def _combine_body(enc_ref, d1_ref, d2_ref, w1_ref, w2_ref, y_ref):
    y_ref[...] = (enc_ref[...]
                  + w1_ref[...] * d1_ref[...]
                  + w2_ref[...] * d2_ref[...])


def _make_sc_gather(n_idx):
    rpw = n_idx // NW
    nch = rpw // SC_CHUNK
    mesh = plsc.VectorSubcoreMesh(core_axis_name="c", subcore_axis_name="s")

    @functools.partial(
        pl.kernel,
        out_type=jax.ShapeDtypeStruct((n_idx, D_MODEL), jnp.float32),
        mesh=mesh,
        scratch_types=[
            pltpu.VMEM((rpw,), jnp.int32),
            pltpu.VMEM((SC_CHUNK, D_MODEL), jnp.float32),
            pltpu.VMEM((SC_CHUNK, D_MODEL), jnp.float32),
            pltpu.SemaphoreType.DMA,
            pltpu.SemaphoreType.DMA,
            pltpu.SemaphoreType.DMA,
            pltpu.SemaphoreType.DMA,
        ],
    )
    def gathr(table_hbm, idx_hbm, out_hbm, idx_v, buf0, buf1,
              gs0, gs1, os0, os1):
        wid = lax.axis_index("s") * NC + lax.axis_index("c")
        base = wid * rpw
        pltpu.sync_copy(idx_hbm.at[pl.ds(base, rpw)], idx_v)
        bufs, gsem, osem = (buf0, buf1), (gs0, gs1), (os0, os1)
        gd = [None, None]
        od = [None, None]
        for c in range(nch):
            b = c & 1
            if od[b] is not None:
                od[b].wait()
            gd[b] = pltpu.async_copy(
                table_hbm.at[idx_v.at[pl.ds(c * SC_CHUNK, SC_CHUNK)]],
                bufs[b], gsem[b])
            if c >= 1:
                pb = (c - 1) & 1
                gd[pb].wait()
                od[pb] = pltpu.async_copy(
                    bufs[pb],
                    out_hbm.at[pl.ds(base + (c - 1) * SC_CHUNK, SC_CHUNK)],
                    osem[pb])
        lb = (nch - 1) & 1
        gd[lb].wait()
        od[lb] = pltpu.async_copy(
            bufs[lb],
            out_hbm.at[pl.ds(base + (nch - 1) * SC_CHUNK, SC_CHUNK)],
            osem[lb])
        for b in range(2):
            if od[b] is not None:
                od[b].wait()

    return gathr


def kernel(x, W_enc, b_enc, W_gate, b_gate, W1, W2):
    f32 = jnp.float32
    wg_pad = jnp.zeros((LOG_PAD, D_MODEL), f32).at[:N_EXPERTS].set(W_gate)
    bg_pad = jnp.full((1, LOG_PAD), -1e30, f32).at[0, :N_EXPERTS].set(b_gate)
    be2 = b_enc.reshape(1, D_MODEL)

    enc, logits = pl.pallas_call(
        _enc_body,
        grid=(B_TOK // BM_ENC,),
        in_specs=[
            pl.BlockSpec((BM_ENC, D_MODEL), lambda i: (i, 0)),
            pl.BlockSpec((D_MODEL, D_MODEL), lambda i: (0, 0)),
            pl.BlockSpec((1, D_MODEL), lambda i: (0, 0)),
            pl.BlockSpec((LOG_PAD, D_MODEL), lambda i: (0, 0)),
            pl.BlockSpec((1, LOG_PAD), lambda i: (0, 0)),
        ],
        out_specs=[
            pl.BlockSpec((BM_ENC, D_MODEL), lambda i: (i, 0)),
            pl.BlockSpec((BM_ENC, LOG_PAD), lambda i: (i, 0)),
        ],
        out_shape=[
            jax.ShapeDtypeStruct((B_TOK, D_MODEL), f32),
            jax.ShapeDtypeStruct((B_TOK, LOG_PAD), f32),
        ],
        compiler_params=pltpu.CompilerParams(
            dimension_semantics=("parallel",)),
    )(x, W_enc, be2, wg_pad, bg_pad)

    pidx, g1, g2, w1k, w2k = pl.pallas_call(
        _route_body,
        grid=(1,),
        in_specs=[pl.BlockSpec((B_TOK, LOG_PAD), lambda i: (0, 0))],
        out_specs=[
            pl.BlockSpec((N_EXPERTS, CAP), lambda i: (0, 0)),
            pl.BlockSpec((B_TOK, 1), lambda i: (0, 0)),
            pl.BlockSpec((B_TOK, 1), lambda i: (0, 0)),
            pl.BlockSpec((B_TOK, 1), lambda i: (0, 0)),
            pl.BlockSpec((B_TOK, 1), lambda i: (0, 0)),
        ],
        out_shape=[
            jax.ShapeDtypeStruct((N_EXPERTS, CAP), jnp.int32),
            jax.ShapeDtypeStruct((B_TOK, 1), jnp.int32),
            jax.ShapeDtypeStruct((B_TOK, 1), jnp.int32),
            jax.ShapeDtypeStruct((B_TOK, 1), f32),
            jax.ShapeDtypeStruct((B_TOK, 1), f32),
        ],
    )(logits)

    xg = _make_sc_gather(N_SLOTS)(enc, pidx.reshape(N_SLOTS))

    delta = pl.pallas_call(
        _ffn_body,
        grid=(N_EXPERTS, D_HIDDEN // BH),
        in_specs=[
            pl.BlockSpec((1, CAP, D_MODEL), lambda e, h: (e, 0, 0)),
            pl.BlockSpec((1, BH, D_MODEL), lambda e, h: (e, h, 0)),
            pl.BlockSpec((1, D_MODEL, BH), lambda e, h: (e, 0, h)),
        ],
        out_specs=pl.BlockSpec((1, CAP, D_MODEL), lambda e, h: (e, 0, 0)),
        out_shape=jax.ShapeDtypeStruct((N_EXPERTS, CAP, D_MODEL), f32),
        compiler_params=pltpu.CompilerParams(
            dimension_semantics=("parallel", "arbitrary")),
    )(xg.reshape(N_EXPERTS, CAP, D_MODEL),
      W1.astype(jnp.bfloat16), W2.astype(jnp.bfloat16))

    g = jnp.concatenate([g1, g2], axis=0).reshape(2 * B_TOK)
    d12 = _make_sc_gather(2 * B_TOK)(delta.reshape(N_SLOTS, D_MODEL), g)

    y = pl.pallas_call(
        _combine_body,
        grid=(B_TOK // BM_ENC,),
        in_specs=[
            pl.BlockSpec((BM_ENC, D_MODEL), lambda i: (i, 0)),
            pl.BlockSpec((BM_ENC, D_MODEL), lambda i: (i, 0)),
            pl.BlockSpec((BM_ENC, D_MODEL), lambda i: (i, 0)),
            pl.BlockSpec((BM_ENC, 1), lambda i: (i, 0)),
            pl.BlockSpec((BM_ENC, 1), lambda i: (i, 0)),
        ],
        out_specs=pl.BlockSpec((BM_ENC, D_MODEL), lambda i: (i, 0)),
        out_shape=jax.ShapeDtypeStruct((B_TOK, D_MODEL), f32),
        compiler_params=pltpu.CompilerParams(
            dimension_semantics=("parallel",)),
    )(enc, d12[:B_TOK], d12[B_TOK:], w1k, w2k)
    return y


# speedup vs baseline: 1.1306x; 1.1306x over previous
import functools

import jax
import jax.numpy as jnp
from jax import lax
from jax.experimental import pallas as pl
from jax.experimental.pallas import tpu as pltpu
from jax.experimental.pallas import tpu_sc as plsc

D_MODEL = 2048
D_HIDDEN = 4096
N_EXPERTS = 8
CAP = 640
B_TOK = 2048
N_SLOTS = N_EXPERTS * CAP

LOG_PAD = 128
BM_ENC = 256
CHUNK_I = 512
BH = 1024

NC, NS = 2, 16
NW = NC * NS
SC_CHUNK = 16


def _enc_body(x_ref, we_ref, be_ref, wg_ref, bg_ref, enc_ref, log_ref):
    enc = lax.dot_general(
        x_ref[...], we_ref[...], (((1,), (1,)), ((), ())),
        preferred_element_type=jnp.float32,
    ) + be_ref[...]
    enc_ref[...] = enc
    log_ref[...] = lax.dot_general(
        enc, wg_ref[...], (((1,), (1,)), ((), ())),
        preferred_element_type=jnp.float32,
    ) + bg_ref[...]


def _route_body(log_ref, pidx_ref, g1_ref, g2_ref, w1_ref, w2_ref):
    logits = log_ref[...]
    col = lax.broadcasted_iota(jnp.int32, (B_TOK, LOG_PAD), 1)
    v1 = jnp.max(logits, axis=1, keepdims=True)
    e1 = jnp.min(jnp.where(logits == v1, col, LOG_PAD), axis=1, keepdims=True)
    l2 = jnp.where(col == e1, -jnp.inf, logits)
    v2 = jnp.max(l2, axis=1, keepdims=True)
    e2 = jnp.min(jnp.where(l2 == v2, col, LOG_PAD), axis=1, keepdims=True)
    t = jnp.exp(v2 - v1)
    denom = (1.0 + t) + 1e-12
    w1 = 1.0 / denom
    w2 = t / denom

    eids = lax.broadcasted_iota(jnp.int32, (B_TOK, N_EXPERTS), 1)
    w_t = jnp.where(eids == e1, w1, 0.0) + jnp.where(eids == e2, w2, 0.0)
    w_all = jnp.transpose(w_t)
    j_row = lax.broadcasted_iota(jnp.int32, (1, B_TOK), 1)
    s_row = lax.broadcasted_iota(jnp.int32, (1, CAP), 1)

    slot1 = jnp.zeros((B_TOK, 1), jnp.int32)
    slot2 = jnp.zeros((B_TOK, 1), jnp.int32)
    k1 = jnp.zeros((B_TOK, 1), jnp.bool_)
    k2 = jnp.zeros((B_TOK, 1), jnp.bool_)

    for e in range(N_EXPERTS):
        w_row = w_all[e:e + 1, :]
        cnt_chunks = []
        for c in range(B_TOK // CHUNK_I):
            i0 = c * CHUNK_I
            w_col = w_t[i0:i0 + CHUNK_I, e:e + 1]
            i_col = lax.broadcasted_iota(jnp.int32, (CHUNK_I, 1), 0) + i0
            beats = (w_row > w_col) | ((w_row == w_col) & (j_row < i_col))
            cnt_chunks.append(jnp.sum(
                jnp.where(beats, 1, 0), axis=1, keepdims=True))
        cnt_e = jnp.concatenate(cnt_chunks, axis=0)
        kept_e = (w_t[:, e:e + 1] > 0.0) & (cnt_e < CAP)
        is1 = e1 == e
        is2 = e2 == e
        slot1 = jnp.where(is1, cnt_e, slot1)
        slot2 = jnp.where(is2, cnt_e, slot2)
        k1 = k1 | (is1 & kept_e)
        k2 = k2 | (is2 & kept_e)
        acc = jnp.zeros((1, CAP), jnp.int32)
        for c in range(B_TOK // CHUNK_I):
            i0 = c * CHUNK_I
            hit = kept_e[i0:i0 + CHUNK_I, :] & (cnt_e[i0:i0 + CHUNK_I, :] == s_row)
            i_col = lax.broadcasted_iota(jnp.int32, (CHUNK_I, 1), 0) + i0
            acc = acc + jnp.sum(jnp.where(hit, i_col, 0), axis=0, keepdims=True)
        nke = jnp.sum(jnp.where(kept_e, 1, 0), axis=0, keepdims=True)
        spread = lax.bitwise_and(s_row + e * CAP, B_TOK - 1)
        pidx_ref[e:e + 1, :] = jnp.where(s_row < nke, acc, spread)

    g1_ref[...] = jnp.where(k1, e1 * CAP + slot1, 0)
    g2_ref[...] = jnp.where(k2, e2 * CAP + slot2, 0)
    w1_ref[...] = jnp.where(k1, w1, 0.0)
    w2_ref[...] = jnp.where(k2, w2, 0.0)


def _ffn_body(xg_ref, w1_ref, w2_ref, out_ref, xb_ref):
    hb = pl.program_id(1)

    @pl.when(hb == 0)
    def _():
        xb_ref[...] = xg_ref[0].astype(jnp.bfloat16)

    h = lax.dot_general(xb_ref[...], w1_ref[0], (((1,), (1,)), ((), ())),
                        preferred_element_type=jnp.float32)
    h = jnp.maximum(h, 0.0).astype(jnp.bfloat16)
    part = lax.dot_general(h, w2_ref[0], (((1,), (1,)), ((), ())),
                           preferred_element_type=jnp.float32)

    @pl.when(hb == 0)
    def _():
        out_ref[0] = part

    @pl.when(hb != 0)
    def _():
        out_ref[0] += part


def _combine_body(enc_ref, d1_ref, d2_ref, w1_ref, w2_ref, y_ref):
    y_ref[...] = (enc_ref[...]
                  + w1_ref[...] * d1_ref[...]
                  + w2_ref[...] * d2_ref[...])


def _make_sc_gather(n_idx):
    rpw = n_idx // NW
    nch = rpw // SC_CHUNK
    mesh = plsc.VectorSubcoreMesh(core_axis_name="c", subcore_axis_name="s")

    @functools.partial(
        pl.kernel,
        out_type=jax.ShapeDtypeStruct((n_idx, D_MODEL), jnp.float32),
        mesh=mesh,
        compiler_params=pltpu.CompilerParams(use_tc_tiling_on_sc=True),
        scratch_types=[
            pltpu.VMEM((rpw,), jnp.int32),
            pltpu.VMEM((SC_CHUNK, D_MODEL), jnp.float32),
            pltpu.VMEM((SC_CHUNK, D_MODEL), jnp.float32),
            pltpu.SemaphoreType.DMA,
            pltpu.SemaphoreType.DMA,
            pltpu.SemaphoreType.DMA,
            pltpu.SemaphoreType.DMA,
        ],
    )
    def gathr(table_hbm, idx_hbm, out_hbm, idx_v, buf0, buf1,
              gs0, gs1, os0, os1):
        wid = lax.axis_index("s") * NC + lax.axis_index("c")
        base = wid * rpw
        pltpu.sync_copy(idx_hbm.at[pl.ds(base, rpw)], idx_v)
        bufs, gsem, osem = (buf0, buf1), (gs0, gs1), (os0, os1)
        gd = [None, None]
        od = [None, None]
        for c in range(nch):
            b = c & 1
            if od[b] is not None:
                od[b].wait()
            gd[b] = pltpu.async_copy(
                table_hbm.at[idx_v.at[pl.ds(c * SC_CHUNK, SC_CHUNK)]],
                bufs[b], gsem[b])
            if c >= 1:
                pb = (c - 1) & 1
                gd[pb].wait()
                od[pb] = pltpu.async_copy(
                    bufs[pb],
                    out_hbm.at[pl.ds(base + (c - 1) * SC_CHUNK, SC_CHUNK)],
                    osem[pb])
        lb = (nch - 1) & 1
        gd[lb].wait()
        od[lb] = pltpu.async_copy(
            bufs[lb],
            out_hbm.at[pl.ds(base + (nch - 1) * SC_CHUNK, SC_CHUNK)],
            osem[lb])
        for b in range(2):
            if od[b] is not None:
                od[b].wait()

    return gathr


def kernel(x, W_enc, b_enc, W_gate, b_gate, W1, W2):
    f32 = jnp.float32
    wg_pad = jnp.zeros((LOG_PAD, D_MODEL), f32).at[:N_EXPERTS].set(W_gate)
    bg_pad = jnp.full((1, LOG_PAD), -1e30, f32).at[0, :N_EXPERTS].set(b_gate)
    be2 = b_enc.reshape(1, D_MODEL)

    enc, logits = pl.pallas_call(
        _enc_body,
        grid=(B_TOK // BM_ENC,),
        in_specs=[
            pl.BlockSpec((BM_ENC, D_MODEL), lambda i: (i, 0)),
            pl.BlockSpec((D_MODEL, D_MODEL), lambda i: (0, 0)),
            pl.BlockSpec((1, D_MODEL), lambda i: (0, 0)),
            pl.BlockSpec((LOG_PAD, D_MODEL), lambda i: (0, 0)),
            pl.BlockSpec((1, LOG_PAD), lambda i: (0, 0)),
        ],
        out_specs=[
            pl.BlockSpec((BM_ENC, D_MODEL), lambda i: (i, 0)),
            pl.BlockSpec((BM_ENC, LOG_PAD), lambda i: (i, 0)),
        ],
        out_shape=[
            jax.ShapeDtypeStruct((B_TOK, D_MODEL), f32),
            jax.ShapeDtypeStruct((B_TOK, LOG_PAD), f32),
        ],
        compiler_params=pltpu.CompilerParams(
            dimension_semantics=("parallel",)),
    )(x, W_enc, be2, wg_pad, bg_pad)

    pidx, g1, g2, w1k, w2k = pl.pallas_call(
        _route_body,
        grid=(1,),
        in_specs=[pl.BlockSpec((B_TOK, LOG_PAD), lambda i: (0, 0))],
        out_specs=[
            pl.BlockSpec((N_EXPERTS, CAP), lambda i: (0, 0)),
            pl.BlockSpec((B_TOK, 1), lambda i: (0, 0)),
            pl.BlockSpec((B_TOK, 1), lambda i: (0, 0)),
            pl.BlockSpec((B_TOK, 1), lambda i: (0, 0)),
            pl.BlockSpec((B_TOK, 1), lambda i: (0, 0)),
        ],
        out_shape=[
            jax.ShapeDtypeStruct((N_EXPERTS, CAP), jnp.int32),
            jax.ShapeDtypeStruct((B_TOK, 1), jnp.int32),
            jax.ShapeDtypeStruct((B_TOK, 1), jnp.int32),
            jax.ShapeDtypeStruct((B_TOK, 1), f32),
            jax.ShapeDtypeStruct((B_TOK, 1), f32),
        ],
    )(logits)

    xg = _make_sc_gather(N_SLOTS)(enc, pidx.reshape(N_SLOTS))

    delta = pl.pallas_call(
        _ffn_body,
        grid=(N_EXPERTS, D_HIDDEN // BH),
        in_specs=[
            pl.BlockSpec((1, CAP, D_MODEL), lambda e, h: (e, 0, 0)),
            pl.BlockSpec((1, BH, D_MODEL), lambda e, h: (e, h, 0)),
            pl.BlockSpec((1, D_MODEL, BH), lambda e, h: (e, 0, h)),
        ],
        out_specs=pl.BlockSpec((1, CAP, D_MODEL), lambda e, h: (e, 0, 0)),
        out_shape=jax.ShapeDtypeStruct((N_EXPERTS, CAP, D_MODEL), f32),
        scratch_shapes=[pltpu.VMEM((CAP, D_MODEL), jnp.bfloat16)],
        compiler_params=pltpu.CompilerParams(
            dimension_semantics=("parallel", "arbitrary")),
    )(xg.reshape(N_EXPERTS, CAP, D_MODEL),
      W1.astype(jnp.bfloat16), W2.astype(jnp.bfloat16))

    g = jnp.concatenate([g1, g2], axis=0).reshape(2 * B_TOK)
    d12 = _make_sc_gather(2 * B_TOK)(delta.reshape(N_SLOTS, D_MODEL), g)

    y = pl.pallas_call(
        _combine_body,
        grid=(B_TOK // BM_ENC,),
        in_specs=[
            pl.BlockSpec((BM_ENC, D_MODEL), lambda i: (i, 0)),
            pl.BlockSpec((BM_ENC, D_MODEL), lambda i: (i, 0)),
            pl.BlockSpec((BM_ENC, D_MODEL), lambda i: (i, 0)),
            pl.BlockSpec((BM_ENC, 1), lambda i: (i, 0)),
            pl.BlockSpec((BM_ENC, 1), lambda i: (i, 0)),
        ],
        out_specs=pl.BlockSpec((BM_ENC, D_MODEL), lambda i: (i, 0)),
        out_shape=jax.ShapeDtypeStruct((B_TOK, D_MODEL), f32),
        compiler_params=pltpu.CompilerParams(
            dimension_semantics=("parallel",)),
    )(enc, d12[:B_TOK], d12[B_TOK:], w1k, w2k)
    return y
